# baseline (speedup 1.0000x reference)
import functools

import numpy as np

import jax
import jax.numpy as jnp
from jax import lax
from jax.experimental import pallas as pl
from jax.experimental.pallas import tpu as pltpu
from jax.experimental.pallas import tpu_sc as plsc

_N = 10000
_E = 320000
_C = 128
_K = 128
_NC = 2
_NS = 16
_NW = _NC * _NS
_EPT = _E // _NW
_CPT = 2 * (-(-_EPT // (2 * _K)))
_EPTP = _CPT * _K
_RPT = 624
_TAIL = _N - _RPT * _NS
_L = 16

_mesh = plsc.VectorSubcoreMesh(core_axis_name="c", subcore_axis_name="s")


@functools.partial(
    pl.kernel,
    out_type=jax.ShapeDtypeStruct((_NC, _N, _C), jnp.float32),
    mesh=_mesh,
    scratch_types=[
        pltpu.VMEM((1, _EPTP // 2), jnp.int32),
        pltpu.VMEM((1, _EPTP // 2), jnp.int32),
        pltpu.VMEM((_K,), jnp.int32),
        pltpu.VMEM((_K,), jnp.int32),
        pltpu.VMEM((_K,), jnp.int32),
        pltpu.VMEM((_K,), jnp.int32),
        pltpu.VMEM((_K, _C), jnp.float32),
        pltpu.VMEM((_K, _C), jnp.float32),
        pltpu.VMEM_SHARED((_N + 8, _C), jnp.float32),
        pltpu.SemaphoreType.DMA,
        pltpu.SemaphoreType.DMA,
    ],
    compiler_params=pltpu.CompilerParams(needs_layout_passes=False),
)
def _aggregate(h_hbm, src_hbm, dst_hbm, out_hbm, src_p, dst_p,
               sidx_a, sidx_b, didx_a, didx_b, rows_a, rows_b,
               acc, sem_a, sem_b):
    cid = lax.axis_index("c")
    sid = lax.axis_index("s")
    wid = cid * jnp.int32(_NS) + sid
    row0 = sid * jnp.int32(_RPT)

    pltpu.sync_copy(src_hbm.at[wid], src_p)
    pltpu.sync_copy(dst_hbm.at[wid], dst_p)

    pltpu.sync_copy(h_hbm.at[pl.ds(row0, _RPT)],
                    acc.at[pl.ds(row0, _RPT)])

    @pl.when(sid == 0)
    def _():
        pltpu.sync_copy(h_hbm.at[pl.ds(_RPT * _NS, _TAIL)],
                        acc.at[pl.ds(_RPT * _NS, _TAIL)])

    plsc.subcore_barrier()

    zero = jnp.int32(0)
    one = jnp.int32(1)
    cmax = jnp.int32(_CPT - 1)
    lo_mask = jnp.int32(0xFFFF)
    sh16 = jnp.int32(16)
    evens = lax.iota(jnp.int32, _L) * jnp.int32(2)

    def unpack_chunk(packed, c, out_ref):
        base = c * jnp.int32(_K // 2)
        for g in range(_K // (2 * _L)):
            w = packed[zero, pl.ds(base + jnp.int32(g * _L), _L)]
            pos = evens + jnp.int32(2 * _L * g)
            plsc.store_scatter(out_ref, [pos], w & lo_mask)
            plsc.store_scatter(out_ref, [pos + one],
                               lax.shift_right_logical(w, sh16))

    def gather_chunk(c, sidx, rows, sem):
        unpack_chunk(src_p, c, sidx)
        return pltpu.async_copy(h_hbm.at[sidx], rows, sem)

    ga0 = gather_chunk(zero, sidx_a, rows_a, sem_a)
    gb0 = gather_chunk(one, sidx_b, rows_b, sem_b)
    del ga0, gb0

    def wait_rows(sidx, rows, sem):
        pltpu.make_async_copy(h_hbm.at[sidx], rows, sem).wait()

    def body(i, _):
        c0 = i * jnp.int32(2)
        n0 = jnp.minimum(c0 + jnp.int32(2), cmax)
        n1 = jnp.minimum(c0 + jnp.int32(3), cmax)
        unpack_chunk(dst_p, c0, didx_a)
        wait_rows(sidx_a, rows_a, sem_a)
        pltpu.sync_copy(rows_a, acc.at[didx_a], add=True)
        gather_chunk(n0, sidx_a, rows_a, sem_a)
        unpack_chunk(dst_p, c0 + one, didx_b)
        wait_rows(sidx_b, rows_b, sem_b)
        pltpu.sync_copy(rows_b, acc.at[didx_b], add=True)
        gather_chunk(n1, sidx_b, rows_b, sem_b)
        return i + one, None

    lax.scan(body, jnp.int32(0), None, length=_CPT // 2)

    wait_rows(sidx_a, rows_a, sem_a)
    wait_rows(sidx_b, rows_b, sem_b)

    plsc.subcore_barrier()

    pltpu.sync_copy(acc.at[pl.ds(row0, _RPT)],
                    out_hbm.at[cid, pl.ds(row0, _RPT)])

    @pl.when(sid == 0)
    def _():
        pltpu.sync_copy(acc.at[pl.ds(_RPT * _NS, _TAIL)],
                        out_hbm.at[cid, pl.ds(_RPT * _NS, _TAIL)])


_BLK = 400


def _zero():
    return jnp.int32(0)


def _mm_body(p_ref, h_ref, w_ref, o_ref):
    a = p_ref[0] + p_ref[1] - h_ref[...]
    o_ref[...] = jnp.maximum(
        lax.dot_general(a, w_ref[...], (((1,), (1,)), ((), ())),
                        preferred_element_type=jnp.float32,
                        precision=lax.Precision.HIGHEST),
        0.0)


def _mm(parts, h, w):
    return pl.pallas_call(
        _mm_body,
        grid=(_N // _BLK,),
        in_specs=[
            pl.BlockSpec((_NC, _BLK, _C), lambda i: (_zero(), i, _zero())),
            pl.BlockSpec((_BLK, _C), lambda i: (i, _zero())),
            pl.BlockSpec((_C, _C), lambda i: (_zero(), _zero())),
        ],
        out_specs=pl.BlockSpec((_BLK, _C), lambda i: (i, _zero())),
        out_shape=jax.ShapeDtypeStruct((_N, _C), jnp.float32),
    )(parts, h, w)


def kernel(x, edge_index, W1, W2):
    x = x.astype(jnp.float32)
    src16 = jnp.pad(edge_index[0].astype(jnp.int16).reshape(_NW, _EPT),
                    ((0, 0), (0, _EPTP - _EPT)), constant_values=np.int16(0))
    dst16 = jnp.pad(edge_index[1].astype(jnp.int16).reshape(_NW, _EPT),
                    ((0, 0), (0, _EPTP - _EPT)), constant_values=np.int16(_N))
    src = lax.bitcast_convert_type(
        src16.reshape(_NW, 1, _EPTP // 2, 2), jnp.int32)
    dst = lax.bitcast_convert_type(
        dst16.reshape(_NW, 1, _EPTP // 2, 2), jnp.int32)
    w1 = W1.astype(jnp.float32)
    w2 = W2.astype(jnp.float32)
    p1 = _aggregate(x, src, dst)
    h1 = _mm(p1, x, w1)
    p2 = _aggregate(h1, src, dst)
    h2 = _mm(p2, h1, w2)
    return h2.astype(jnp.float64)


# speedup vs baseline: 2.1927x; 2.1927x over previous
import functools

import numpy as np

import jax
import jax.numpy as jnp
from jax import lax
from jax.experimental import pallas as pl
from jax.experimental.pallas import tpu as pltpu
from jax.experimental.pallas import tpu_sc as plsc

_N = 10000
_E = 320000
_C = 128
_K = 128
_NC = 2
_NS = 16
_NW = _NC * _NS
_EPT = _E // _NW
_CPT = -(-_EPT // _K)
_EPTP = _CPT * _K
_RPT = 624
_TAIL = _N - _RPT * _NS

_mesh = plsc.VectorSubcoreMesh(core_axis_name="c", subcore_axis_name="s")


@functools.partial(
    pl.kernel,
    out_type=jax.ShapeDtypeStruct((_NC, _N, _C), jnp.float32),
    mesh=_mesh,
    scratch_types=[
        pltpu.VMEM((1, _EPTP), jnp.int32),
        pltpu.VMEM((1, _EPTP), jnp.int32),
        pltpu.VMEM((_K, _C), jnp.float32),
        pltpu.VMEM_SHARED((_N + 8, _C), jnp.float32),
        pltpu.SemaphoreType.DMA,
    ],
)
def _aggregate(h_hbm, src_hbm, dst_hbm, out_hbm, src_v, dst_v, rows_v, acc, sem):
    cid = lax.axis_index("c")
    sid = lax.axis_index("s")
    wid = cid * jnp.int32(_NS) + sid
    row0 = sid * jnp.int32(_RPT)

    pltpu.sync_copy(src_hbm.at[wid], src_v)
    pltpu.sync_copy(dst_hbm.at[wid], dst_v)

    pltpu.sync_copy(h_hbm.at[pl.ds(row0, _RPT)],
                    acc.at[pl.ds(row0, _RPT)])

    @pl.when(sid == 0)
    def _():
        pltpu.sync_copy(h_hbm.at[pl.ds(_RPT * _NS, _TAIL)],
                        acc.at[pl.ds(_RPT * _NS, _TAIL)])

    plsc.subcore_barrier()

    zero = jnp.int32(0)

    def body(j, _):
        off = j * jnp.int32(_K)
        pltpu.async_copy(h_hbm.at[src_v.at[zero, pl.ds(off, _K)]], rows_v, sem).wait()
        pltpu.sync_copy(rows_v, acc.at[dst_v.at[zero, pl.ds(off, _K)]], add=True)
        return j + jnp.int32(1), None

    lax.scan(body, jnp.int32(0), None, length=_CPT, unroll=4)

    plsc.subcore_barrier()

    pltpu.sync_copy(acc.at[pl.ds(row0, _RPT)],
                    out_hbm.at[cid, pl.ds(row0, _RPT)])

    @pl.when(sid == 0)
    def _():
        pltpu.sync_copy(acc.at[pl.ds(_RPT * _NS, _TAIL)],
                        out_hbm.at[cid, pl.ds(_RPT * _NS, _TAIL)])


_BLK = 400


def _zero():
    return jnp.int32(0)


def _mm_body(p_ref, h_ref, w_ref, o_ref):
    a = p_ref[0] + p_ref[1] - h_ref[...]
    o_ref[...] = jnp.maximum(
        lax.dot_general(a, w_ref[...], (((1,), (1,)), ((), ())),
                        preferred_element_type=jnp.float32,
                        precision=lax.Precision.HIGHEST),
        0.0)


def _mm(parts, h, w):
    return pl.pallas_call(
        _mm_body,
        grid=(_N // _BLK,),
        in_specs=[
            pl.BlockSpec((_NC, _BLK, _C), lambda i: (_zero(), i, _zero())),
            pl.BlockSpec((_BLK, _C), lambda i: (i, _zero())),
            pl.BlockSpec((_C, _C), lambda i: (_zero(), _zero())),
        ],
        out_specs=pl.BlockSpec((_BLK, _C), lambda i: (i, _zero())),
        out_shape=jax.ShapeDtypeStruct((_N, _C), jnp.float32),
    )(parts, h, w)


def kernel(x, edge_index, W1, W2):
    x = x.astype(jnp.float32)
    pad = ((0, 0), (0, _EPTP - _EPT))
    src = jnp.pad(edge_index[0].astype(jnp.int32).reshape(_NW, _EPT), pad,
                  constant_values=0).reshape(_NW, 1, _EPTP)
    dst = jnp.pad(edge_index[1].astype(jnp.int32).reshape(_NW, _EPT), pad,
                  constant_values=_N).reshape(_NW, 1, _EPTP)
    w1 = W1.astype(jnp.float32)
    w2 = W2.astype(jnp.float32)
    p1 = _aggregate(x, src, dst)
    h1 = _mm(p1, x, w1)
    p2 = _aggregate(h1, src, dst)
    h2 = _mm(p2, h1, w2)
    return h2.astype(jnp.float64)


# speedup vs baseline: 2.1957x; 1.0014x over previous
import functools

import numpy as np

import jax
import jax.numpy as jnp
from jax import lax
from jax.experimental import pallas as pl
from jax.experimental.pallas import tpu as pltpu
from jax.experimental.pallas import tpu_sc as plsc

_N = 10000
_E = 320000
_C = 128
_K = 128
_NC = 2
_NS = 16
_NW = _NC * _NS
_EPT = _E // _NW
_CPT = -(-_EPT // _K)
_EPTP = _CPT * _K
_RPT = 624
_TAIL = _N - _RPT * _NS

_mesh = plsc.VectorSubcoreMesh(core_axis_name="c", subcore_axis_name="s")


@functools.partial(
    pl.kernel,
    out_type=jax.ShapeDtypeStruct((_NC, _N, _C), jnp.float32),
    mesh=_mesh,
    scratch_types=[
        pltpu.VMEM((1, _EPTP), jnp.int32),
        pltpu.VMEM((1, _EPTP), jnp.int32),
        pltpu.VMEM((_K, _C), jnp.float32),
        pltpu.VMEM_SHARED((_N + 8, _C), jnp.float32),
        pltpu.SemaphoreType.DMA,
    ],
)
def _aggregate(h_hbm, src_hbm, dst_hbm, out_hbm, src_v, dst_v, rows_v, acc, sem):
    cid = lax.axis_index("c")
    sid = lax.axis_index("s")
    wid = cid * jnp.int32(_NS) + sid
    row0 = sid * jnp.int32(_RPT)

    pltpu.sync_copy(src_hbm.at[wid], src_v)
    pltpu.sync_copy(dst_hbm.at[wid], dst_v)

    pltpu.sync_copy(h_hbm.at[pl.ds(row0, _RPT)],
                    acc.at[pl.ds(row0, _RPT)])

    @pl.when(sid == 0)
    def _():
        pltpu.sync_copy(h_hbm.at[pl.ds(_RPT * _NS, _TAIL)],
                        acc.at[pl.ds(_RPT * _NS, _TAIL)])

    plsc.subcore_barrier()

    zero = jnp.int32(0)

    def body(j, _):
        off = j * jnp.int32(_K)
        pltpu.async_copy(h_hbm.at[src_v.at[zero, pl.ds(off, _K)]], rows_v, sem).wait()
        pltpu.sync_copy(rows_v, acc.at[dst_v.at[zero, pl.ds(off, _K)]], add=True)
        return j + jnp.int32(1), None

    lax.scan(body, jnp.int32(0), None, length=_CPT)

    plsc.subcore_barrier()

    pltpu.sync_copy(acc.at[pl.ds(row0, _RPT)],
                    out_hbm.at[cid, pl.ds(row0, _RPT)])

    @pl.when(sid == 0)
    def _():
        pltpu.sync_copy(acc.at[pl.ds(_RPT * _NS, _TAIL)],
                        out_hbm.at[cid, pl.ds(_RPT * _NS, _TAIL)])


_BLK = 400


def _zero():
    return jnp.int32(0)


def _mm_body(p_ref, h_ref, w_ref, o_ref):
    a = p_ref[0] + p_ref[1] - h_ref[...]
    o_ref[...] = jnp.maximum(
        lax.dot_general(a, w_ref[...], (((1,), (1,)), ((), ())),
                        preferred_element_type=jnp.float32,
                        precision=lax.Precision.HIGHEST),
        0.0)


def _mm(parts, h, w):
    return pl.pallas_call(
        _mm_body,
        grid=(_N // _BLK,),
        in_specs=[
            pl.BlockSpec((_NC, _BLK, _C), lambda i: (_zero(), i, _zero())),
            pl.BlockSpec((_BLK, _C), lambda i: (i, _zero())),
            pl.BlockSpec((_C, _C), lambda i: (_zero(), _zero())),
        ],
        out_specs=pl.BlockSpec((_BLK, _C), lambda i: (i, _zero())),
        out_shape=jax.ShapeDtypeStruct((_N, _C), jnp.float32),
    )(parts, h, w)


def kernel(x, edge_index, W1, W2):
    x = x.astype(jnp.float32)
    pad = ((0, 0), (0, _EPTP - _EPT))
    src = jnp.pad(edge_index[0].astype(jnp.int32).reshape(_NW, _EPT), pad,
                  constant_values=0).reshape(_NW, 1, _EPTP)
    dst = jnp.pad(edge_index[1].astype(jnp.int32).reshape(_NW, _EPT), pad,
                  constant_values=_N).reshape(_NW, 1, _EPTP)
    w1 = W1.astype(jnp.float32)
    w2 = W2.astype(jnp.float32)
    p1 = _aggregate(x, src, dst)
    h1 = _mm(p1, x, w1)
    p2 = _aggregate(h1, src, dst)
    h2 = _mm(p2, h1, w2)
    return h2.astype(jnp.float64)
